# Initial kernel scaffold; baseline (speedup 1.0000x reference)
import jax
import jax.numpy as jnp
from jax.experimental import pallas as pl


def kernel(x, dendrites, kernel, dendriticW, bias):
    raise NotImplementedError("write your pallas kernel here")


# speedup vs baseline: 8.4470x; 8.4470x over previous
import functools

import jax
import jax.numpy as jnp
from jax import lax
from jax.experimental import pallas as pl
from jax.experimental.pallas import tpu as pltpu
from jax.experimental.pallas import tpu_sc as plsc

B, C, U, D, S = 1024, 256, 128, 16, 16
NC, NS = 2, 16
NW = NC * NS
UPW = U // NW


def _sc_scatter_kernel(duds_hbm, dwt_hbm, wt_hbm, idx_v, dw_v, w_v):
    wid = lax.axis_index("s") * NC + lax.axis_index("c")
    pltpu.sync_copy(duds_hbm.at[pl.ds(wid * (UPW * D * S), UPW * D * S)], idx_v)
    pltpu.sync_copy(dwt_hbm.at[pl.ds(wid * (UPW * S), UPW * S)], dw_v)
    for j in range(UPW):
        w16 = dw_v[pl.ds(j * S, 16)]
        for d in range(D):
            idx = idx_v[pl.ds(j * D * S + d * S, 16)]
            plsc.store_scatter(w_v, [idx + j * C], w16)
    pltpu.sync_copy(w_v, wt_hbm.at[pl.ds(wid * (UPW * C), UPW * C)])


@functools.partial(
    pl.kernel,
    mesh=plsc.VectorSubcoreMesh(core_axis_name="c", subcore_axis_name="s"),
    out_type=jax.ShapeDtypeStruct((U * C,), jnp.float32),
    scratch_types=[
        pltpu.VMEM((UPW * D * S,), jnp.int32),
        pltpu.VMEM((UPW * S,), jnp.float32),
        pltpu.VMEM((UPW * C,), jnp.float32),
    ],
    compiler_params=pltpu.CompilerParams(needs_layout_passes=False),
)
def _sc_scatter(duds_hbm, dwt_hbm, wt_hbm, idx_v, dw_v, w_v):
    _sc_scatter_kernel(duds_hbm, dwt_hbm, wt_hbm, idx_v, dw_v, w_v)


def _tc_matmul_kernel(x_ref, wt_ref, kw_ref, b_ref, o_ref):
    acc = lax.dot_general(
        x_ref[:], wt_ref[:], (((1,), (1,)), ((), ())),
        preferred_element_type=jnp.float32)
    o_ref[:] = acc * kw_ref[:] + b_ref[:]


def _tc_matmul(x, wt, kw, b2):
    return pl.pallas_call(
        _tc_matmul_kernel,
        out_shape=jax.ShapeDtypeStruct((B, U), jnp.float32),
    )(x, wt, kw, b2)


def kernel(x, dendrites, kernel, dendriticW, bias):
    duds = jnp.transpose(dendrites, (2, 0, 1)).reshape(U * D * S)
    dwt = jnp.transpose(dendriticW).reshape(U * S)
    wt = _sc_scatter(duds, dwt).reshape(U, C)
    return _tc_matmul(x, wt, kernel, bias.reshape(1, U))
